# Initial kernel scaffold; baseline (speedup 1.0000x reference)
import functools

import jax
import jax.numpy as jnp
from jax import lax
from jax.experimental import pallas as pl
from jax.experimental.pallas import tpu as pltpu
from jax.experimental.pallas import tpu_sc as plsc

_N = 10000
_E = 320000
_D = 128
_G = 64

_NC = 2
_NS = 16
_EPC = _E // _NC
_EPW = _E // (_NC * _NS)
_CH = 128
_NFULL = _EPW // _CH
_TAIL = _EPW - _NFULL * _CH
_RPS = _N // _NS
_ZR = 125
_CW = 16

_F32 = jnp.float32


def _make_sc_msgpass(with_cnt):
  mesh = plsc.VectorSubcoreMesh(core_axis_name="c", subcore_axis_name="s")
  out_type = [jax.ShapeDtypeStruct((_NC, _N, _D), _F32)]
  scratch = [
      pltpu.VMEM_SHARED((_N, _D), _F32),
      pltpu.VMEM((_ZR, _D), _F32),
      pltpu.VMEM((_CH,), jnp.int32),
      pltpu.VMEM((_CH,), jnp.int32),
      pltpu.VMEM((_CH, _D), _F32),
      pltpu.VMEM((_TAIL,), jnp.int32),
      pltpu.VMEM((_TAIL,), jnp.int32),
      pltpu.VMEM((_TAIL, _D), _F32),
  ]
  if with_cnt:
    out_type.append(jax.ShapeDtypeStruct((_NC, _N, _CW), _F32))
    scratch += [
        pltpu.VMEM_SHARED((_N, _CW), _F32),
        pltpu.VMEM((_ZR, _CW), _F32),
        pltpu.VMEM((_CH, _CW), _F32),
        pltpu.VMEM((_TAIL, _CW), _F32),
    ]

  def body(x_hbm, src_hbm, dst_hbm, *rest):
    if with_cnt:
      (agg_hbm, cnt_hbm, agg_sh, zbuf, src_v, dst_v, rows_v,
       srct_v, dstt_v, rowst_v, cnt_sh, zcnt, ones_v, onest_v) = rest
    else:
      (agg_hbm, agg_sh, zbuf, src_v, dst_v, rows_v,
       srct_v, dstt_v, rowst_v) = rest
    c = lax.axis_index("c")
    s = lax.axis_index("s")

    @pl.loop(0, _ZR)
    def _(i):
      @pl.loop(0, _D // 16)
      def _(j):
        zbuf.at[i, pl.ds(j * 16, 16)][...] = jnp.zeros((16,), _F32)

    if with_cnt:
      @pl.loop(0, _ZR)
      def _(i):
        zcnt.at[i][...] = jnp.zeros((_CW,), _F32)

      @pl.loop(0, _CH)
      def _(i):
        ones_v.at[i][...] = jnp.ones((_CW,), _F32)

      @pl.loop(0, _TAIL)
      def _(i):
        onest_v.at[i][...] = jnp.ones((_CW,), _F32)

    @pl.loop(0, _RPS // _ZR)
    def _(k):
      pltpu.sync_copy(zbuf, agg_sh.at[pl.ds(s * _RPS + k * _ZR, _ZR)])

    if with_cnt:
      @pl.loop(0, _RPS // _ZR)
      def _(k):
        pltpu.sync_copy(zcnt, cnt_sh.at[pl.ds(s * _RPS + k * _ZR, _ZR)])

    plsc.subcore_barrier()

    base0 = c * _EPC + s * _EPW

    @pl.loop(0, _NFULL)
    def _(i):
      b = base0 + i * _CH
      pltpu.sync_copy(src_hbm.at[pl.ds(b, _CH)], src_v)
      pltpu.sync_copy(dst_hbm.at[pl.ds(b, _CH)], dst_v)
      pltpu.sync_copy(x_hbm.at[src_v], rows_v)
      pltpu.sync_copy(rows_v, agg_sh.at[dst_v], add=True)
      if with_cnt:
        pltpu.sync_copy(ones_v, cnt_sh.at[dst_v], add=True)

    bt = base0 + _NFULL * _CH
    pltpu.sync_copy(src_hbm.at[pl.ds(bt, _TAIL)], srct_v)
    pltpu.sync_copy(dst_hbm.at[pl.ds(bt, _TAIL)], dstt_v)
    pltpu.sync_copy(x_hbm.at[srct_v], rowst_v)
    pltpu.sync_copy(rowst_v, agg_sh.at[dstt_v], add=True)
    if with_cnt:
      pltpu.sync_copy(onest_v, cnt_sh.at[dstt_v], add=True)

    plsc.subcore_barrier()

    pltpu.sync_copy(agg_sh.at[pl.ds(s * _RPS, _RPS)],
                    agg_hbm.at[c, pl.ds(s * _RPS, _RPS)])
    if with_cnt:
      pltpu.sync_copy(cnt_sh.at[pl.ds(s * _RPS, _RPS)],
                      cnt_hbm.at[c, pl.ds(s * _RPS, _RPS)])

  return pl.kernel(body, out_type=tuple(out_type), mesh=mesh,
                   scratch_types=scratch)


_sc_layer1 = _make_sc_msgpass(True)
_sc_layer2 = _make_sc_msgpass(False)

_HI = lax.Precision.HIGHEST


def _tc1_body(agg_ref, cnt_ref, x_ref, w1l_ref, b1l_ref, w1r_ref, h_ref):
  cnt = cnt_ref[0][:, :1] + cnt_ref[1][:, :1]
  mean = (agg_ref[0] + agg_ref[1]) / jnp.maximum(cnt, 1.0)
  h = (jnp.dot(mean, w1l_ref[...], precision=_HI, preferred_element_type=_F32)
       + jnp.dot(x_ref[...], w1r_ref[...], precision=_HI,
                 preferred_element_type=_F32)
       + b1l_ref[...])
  h_ref[...] = jnp.maximum(h, 0.0)


def _tc2_body(agg_ref, cnt_ref, h_ref, w2l_ref, b2l_ref, w2r_ref, batch_ref,
              pooled_ref, h2_ref):
  cnt = cnt_ref[0][:, :1] + cnt_ref[1][:, :1]
  mean = (agg_ref[0] + agg_ref[1]) / jnp.maximum(cnt, 1.0)
  h2 = (jnp.dot(mean, w2l_ref[...], precision=_HI, preferred_element_type=_F32)
        + jnp.dot(h_ref[...], w2r_ref[...], precision=_HI,
                  preferred_element_type=_F32)
        + b2l_ref[...])
  h2_ref[...] = h2
  sel = (lax.broadcasted_iota(jnp.int32, (_G, _N), 0)
         == batch_ref[...]).astype(_F32)
  psum = jnp.dot(sel, h2, precision=_HI, preferred_element_type=_F32)
  cg = jnp.sum(sel, axis=1, keepdims=True)
  pooled_ref[...] = psum / jnp.maximum(cg, 1.0)


def kernel(x, edge_index, batch, W1l, b1l, W1r, W2l, b2l, W2r):
  src = edge_index[0]
  dst = edge_index[1]
  agg1, cnt = _sc_layer1(x, src, dst)
  h = pl.pallas_call(
      _tc1_body,
      out_shape=jax.ShapeDtypeStruct((_N, _D), _F32),
  )(agg1, cnt, x, W1l, b1l.reshape(1, _D), W1r)
  agg2 = _sc_layer2(h, src, dst)
  pooled, h2 = pl.pallas_call(
      _tc2_body,
      out_shape=(jax.ShapeDtypeStruct((_G, _D), _F32),
                 jax.ShapeDtypeStruct((_N, _D), _F32)),
  )(agg2, cnt, h, W2l, b2l.reshape(1, _D), W2r, batch.reshape(1, _N))
  return (pooled, h2)


# speedup vs baseline: 5.5553x; 5.5553x over previous
import dataclasses
import functools

import jax
import jax.numpy as jnp
from jax import lax
from jax.experimental import pallas as pl
from jax.experimental.pallas import tpu as pltpu
from jax.experimental.pallas import tpu_sc as plsc

_N = 10000
_E = 320000
_D = 128
_G = 64

_NC = 2
_NS = 16
_NW = _NC * _NS
_EPW = _E // _NW
_CH = 80
_NCHUNK = _EPW // _CH
_RSTRIDE = 624
_RWIN = 640
_ZR = 16
_L = 16

_F32 = jnp.float32


def _make_sc_msgpass(with_cnt):
  mesh = plsc.VectorSubcoreMesh(core_axis_name="c", subcore_axis_name="s")
  out_type = [jax.ShapeDtypeStruct((_NC, _N, _D), _F32)]
  scratch = [
      pltpu.VMEM_SHARED((_N, _D), _F32),
      pltpu.VMEM((_ZR, _D), _F32),
      pltpu.VMEM((_CH,), jnp.int32),
      pltpu.VMEM((_CH,), jnp.int32),
      pltpu.VMEM((_CH, _D), _F32),
  ]
  if with_cnt:
    out_type.append(jax.ShapeDtypeStruct((_NC, _NS, _N), _F32))
    scratch.append(pltpu.VMEM((_N,), _F32))
  cp = pltpu.CompilerParams()
  if "needs_layout_passes" in pltpu.CompilerParams.__dataclass_fields__:
    cp = dataclasses.replace(cp, needs_layout_passes=False)

  def body(x_hbm, src_hbm, dst_hbm, *rest):
    if with_cnt:
      (agg_hbm, cnt_hbm, agg_sh, zbuf, src_v, dst_v, rows_v, hist) = rest
    else:
      (agg_hbm, agg_sh, zbuf, src_v, dst_v, rows_v) = rest
    c = lax.axis_index("c")
    s = lax.axis_index("s")

    @pl.loop(0, _ZR)
    def _(i):
      @pl.loop(0, _D // _L)
      def _(j):
        zbuf.at[i, pl.ds(j * _L, _L)][...] = jnp.zeros((_L,), _F32)

    if with_cnt:
      @pl.loop(0, _N // _L)
      def _(i):
        hist.at[pl.ds(i * _L, _L)][...] = jnp.zeros((_L,), _F32)

    rbase = s * _RSTRIDE

    @pl.loop(0, _RWIN // _ZR)
    def _(k):
      pltpu.sync_copy(zbuf, agg_sh.at[pl.ds(rbase + k * _ZR, _ZR)])

    plsc.subcore_barrier()

    wid = c * _NS + s
    base0 = wid * _EPW

    @pl.loop(0, _NCHUNK)
    def _(i):
      b = base0 + i * _CH
      pltpu.sync_copy(src_hbm.at[pl.ds(b, _CH)], src_v)
      pltpu.sync_copy(dst_hbm.at[pl.ds(b, _CH)], dst_v)
      pltpu.sync_copy(x_hbm.at[src_v], rows_v)
      pltpu.sync_copy(rows_v, agg_sh.at[dst_v], add=True)
      if with_cnt:
        @pl.loop(0, _CH // _L)
        def _(k):
          ii = dst_v.at[pl.ds(k * _L, _L)][...]
          plsc.addupdate_scatter(hist, [ii], jnp.ones((_L,), _F32))

    plsc.subcore_barrier()

    pltpu.sync_copy(agg_sh.at[pl.ds(rbase, _RWIN)],
                    agg_hbm.at[c, pl.ds(rbase, _RWIN)])
    if with_cnt:
      pltpu.sync_copy(hist, cnt_hbm.at[c, s])

  return pl.kernel(body, out_type=tuple(out_type), mesh=mesh,
                   scratch_types=scratch, compiler_params=cp)


_sc_layer1 = _make_sc_msgpass(True)
_sc_layer2 = _make_sc_msgpass(False)

_HI = lax.Precision.HIGHEST
_R = 2000
_NB = _N // _R


def _cnt_col(cnt_ref):
  cnt = cnt_ref[:, 0, 0, :]
  return lax.dot_general(cnt, jnp.ones((_NW, 1), _F32),
                         (((0,), (0,)), ((), ())),
                         precision=_HI, preferred_element_type=_F32)


def _tc1_body(agg_ref, cnt_ref, x_ref, w1l_ref, b1l_ref, w1r_ref, h_ref):
  cnt = _cnt_col(cnt_ref)
  mean = (agg_ref[0] + agg_ref[1]) / jnp.maximum(cnt, 1.0)
  h = (jnp.dot(mean, w1l_ref[...], precision=_HI, preferred_element_type=_F32)
       + jnp.dot(x_ref[...], w1r_ref[...], precision=_HI,
                 preferred_element_type=_F32)
       + b1l_ref[...])
  h_ref[...] = jnp.maximum(h, 0.0)


def _tc2_body(agg_ref, cnt_ref, h_ref, w2l_ref, b2l_ref, w2r_ref, batch_ref,
              pooled_ref, h2_ref, cg_ref):
  i = pl.program_id(0)
  cnt = _cnt_col(cnt_ref)
  mean = (agg_ref[0] + agg_ref[1]) / jnp.maximum(cnt, 1.0)
  h2 = (jnp.dot(mean, w2l_ref[...], precision=_HI, preferred_element_type=_F32)
        + jnp.dot(h_ref[...], w2r_ref[...], precision=_HI,
                  preferred_element_type=_F32)
        + b2l_ref[...])
  h2_ref[...] = h2
  sel = (lax.broadcasted_iota(jnp.int32, (_G, _R), 0)
         == batch_ref[0]).astype(_F32)
  psum = jnp.dot(sel, h2, precision=_HI, preferred_element_type=_F32)
  cg = jnp.sum(sel, axis=1, keepdims=True)

  @pl.when(i == 0)
  def _():
    pooled_ref[...] = jnp.zeros_like(pooled_ref)
    cg_ref[...] = jnp.zeros_like(cg_ref)

  pooled_ref[...] += psum
  cg_ref[...] += cg

  @pl.when(i == _NB - 1)
  def _():
    pooled_ref[...] = pooled_ref[...] / jnp.maximum(cg_ref[...], 1.0)


def kernel(x, edge_index, batch, W1l, b1l, W1r, W2l, b2l, W2r):
  src = edge_index[0]
  dst = edge_index[1]
  agg1, cnt = _sc_layer1(x, src, dst)
  w_spec = pl.BlockSpec((_D, _D), lambda i: (0, 0))
  b_spec = pl.BlockSpec((1, _D), lambda i: (0, 0))
  agg_spec = pl.BlockSpec((_NC, _R, _D), lambda i: (0, i, 0))
  cnt_spec = pl.BlockSpec((_NW, 1, 1, _R), lambda i: (0, i, 0, 0))
  cnt4 = cnt.reshape(_NW, _NB, 1, _R)
  row_spec = pl.BlockSpec((_R, _D), lambda i: (i, 0))
  h = pl.pallas_call(
      _tc1_body,
      grid=(_NB,),
      in_specs=[agg_spec, cnt_spec, row_spec, w_spec, b_spec, w_spec],
      out_specs=row_spec,
      out_shape=jax.ShapeDtypeStruct((_N, _D), _F32),
  )(agg1, cnt4, x, W1l, b1l.reshape(1, _D), W1r)
  (agg2,) = _sc_layer2(h, src, dst)
  pooled, h2 = pl.pallas_call(
      _tc2_body,
      grid=(_NB,),
      in_specs=[agg_spec, cnt_spec, row_spec, w_spec, b_spec, w_spec,
                pl.BlockSpec((1, 1, _R), lambda i: (i, 0, 0))],
      out_specs=(pl.BlockSpec((_G, _D), lambda i: (0, 0)), row_spec),
      out_shape=(jax.ShapeDtypeStruct((_G, _D), _F32),
                 jax.ShapeDtypeStruct((_N, _D), _F32)),
      scratch_shapes=[pltpu.VMEM((_G, 1), _F32)],
  )(agg2, cnt4, h, W2l, b2l.reshape(1, _D), W2r, batch.reshape(_NB, 1, _R))
  return (pooled, h2)


# speedup vs baseline: 11.5262x; 2.0748x over previous
import dataclasses
import functools

import jax
import jax.numpy as jnp
from jax import lax
from jax.experimental import pallas as pl
from jax.experimental.pallas import tpu as pltpu
from jax.experimental.pallas import tpu_sc as plsc

_N = 10000
_E = 320000
_D = 128
_G = 64

_NC = 2
_NS = 16
_NW = _NC * _NS
_EPW = _E // _NW
_CH = 80
_GC = 25
_NG = _EPW // (_GC * _CH)
_RSTRIDE = 624
_RWIN = 640
_ZR = 16
_L = 16

_F32 = jnp.float32


def _make_sc_msgpass(with_cnt):
  mesh = plsc.VectorSubcoreMesh(core_axis_name="c", subcore_axis_name="s")
  out_type = [jax.ShapeDtypeStruct((_NC, _N, _D), _F32)]
  scratch = [
      pltpu.VMEM_SHARED((_N, _D), _F32),
      pltpu.VMEM((_ZR, _D), _F32),
      pltpu.VMEM((_GC, _CH), jnp.int32),
      pltpu.VMEM((_GC, _CH), jnp.int32),
      pltpu.VMEM((_CH, _D), _F32),
      pltpu.VMEM((_CH, _D), _F32),
      pltpu.SemaphoreType.DMA,
      pltpu.SemaphoreType.DMA,
  ]
  if with_cnt:
    out_type.append(jax.ShapeDtypeStruct((_NC, _NS, _N), _F32))
    scratch.append(pltpu.VMEM((_N,), _F32))
  cp = pltpu.CompilerParams()
  if "needs_layout_passes" in pltpu.CompilerParams.__dataclass_fields__:
    cp = dataclasses.replace(cp, needs_layout_passes=False)

  def body(x_hbm, src_hbm, dst_hbm, *rest):
    if with_cnt:
      (agg_hbm, cnt_hbm, agg_sh, zbuf, src_v, dst_v, rows_a, rows_b,
       sem_a, sem_b, hist) = rest
    else:
      (agg_hbm, agg_sh, zbuf, src_v, dst_v, rows_a, rows_b,
       sem_a, sem_b) = rest
    c = lax.axis_index("c")
    s = lax.axis_index("s")

    @pl.loop(0, _ZR)
    def _(i):
      @pl.loop(0, _D // _L)
      def _(j):
        zbuf.at[i, pl.ds(j * _L, _L)][...] = jnp.zeros((_L,), _F32)

    if with_cnt:
      @pl.loop(0, _N // _L)
      def _(i):
        hist.at[pl.ds(i * _L, _L)][...] = jnp.zeros((_L,), _F32)

    rbase = s * _RSTRIDE

    @pl.loop(0, _RWIN // _ZR)
    def _(k):
      pltpu.sync_copy(zbuf, agg_sh.at[pl.ds(rbase + k * _ZR, _ZR)])

    plsc.subcore_barrier()

    wid = c * _NS + s

    def start_gather(i, buf, sem):
      pltpu.async_copy(x_hbm.at[src_v.at[i]], buf, sem)

    def finish_chunk(i, buf, sem):
      pltpu.make_async_copy(x_hbm.at[src_v.at[i]], buf, sem).wait()
      pltpu.sync_copy(buf, agg_sh.at[dst_v.at[i]], add=True)
      if with_cnt:
        @pl.loop(0, _CH // _L)
        def _(k):
          ii = dst_v.at[i, pl.ds(k * _L, _L)][...]
          plsc.addupdate_scatter(hist, [ii], jnp.ones((_L,), _F32))

    @pl.loop(0, _NG)
    def _(g):
      pltpu.sync_copy(src_hbm.at[wid, g], src_v)
      pltpu.sync_copy(dst_hbm.at[wid, g], dst_v)
      start_gather(0, rows_a, sem_a)

      @pl.loop(0, (_GC - 1) // 2)
      def _(jj):
        i0 = 2 * jj
        start_gather(i0 + 1, rows_b, sem_b)
        finish_chunk(i0, rows_a, sem_a)
        start_gather(i0 + 2, rows_a, sem_a)
        finish_chunk(i0 + 1, rows_b, sem_b)

      finish_chunk(_GC - 1, rows_a, sem_a)

    plsc.subcore_barrier()

    pltpu.sync_copy(agg_sh.at[pl.ds(rbase, _RWIN)],
                    agg_hbm.at[c, pl.ds(rbase, _RWIN)])
    if with_cnt:
      pltpu.sync_copy(hist, cnt_hbm.at[c, s])

  kern = pl.kernel(body, out_type=tuple(out_type), mesh=mesh,
                   scratch_types=scratch, compiler_params=cp)

  def call(x, src, dst):
    return kern(x, src.reshape(_NW, _NG, _GC, _CH),
                dst.reshape(_NW, _NG, _GC, _CH))

  return call


_sc_layer1 = _make_sc_msgpass(True)
_sc_layer2 = _make_sc_msgpass(False)

_HI = lax.Precision.HIGHEST
_R = 2000
_NB = _N // _R


def _cnt_col(cnt_ref):
  cnt = cnt_ref[:, 0, 0, :]
  return lax.dot_general(cnt, jnp.ones((_NW, 1), _F32),
                         (((0,), (0,)), ((), ())),
                         precision=_HI, preferred_element_type=_F32)


def _tc1_body(agg_ref, cnt_ref, x_ref, w1l_ref, b1l_ref, w1r_ref, h_ref):
  cnt = _cnt_col(cnt_ref)
  mean = (agg_ref[0] + agg_ref[1]) / jnp.maximum(cnt, 1.0)
  h = (jnp.dot(mean, w1l_ref[...], precision=_HI, preferred_element_type=_F32)
       + jnp.dot(x_ref[...], w1r_ref[...], precision=_HI,
                 preferred_element_type=_F32)
       + b1l_ref[...])
  h_ref[...] = jnp.maximum(h, 0.0)


def _tc2_body(agg_ref, cnt_ref, h_ref, w2l_ref, b2l_ref, w2r_ref, batch_ref,
              pooled_ref, h2_ref, cg_ref):
  i = pl.program_id(0)
  cnt = _cnt_col(cnt_ref)
  mean = (agg_ref[0] + agg_ref[1]) / jnp.maximum(cnt, 1.0)
  h2 = (jnp.dot(mean, w2l_ref[...], precision=_HI, preferred_element_type=_F32)
        + jnp.dot(h_ref[...], w2r_ref[...], precision=_HI,
                  preferred_element_type=_F32)
        + b2l_ref[...])
  h2_ref[...] = h2
  sel = (lax.broadcasted_iota(jnp.int32, (_G, _R), 0)
         == batch_ref[0]).astype(_F32)
  psum = jnp.dot(sel, h2, precision=_HI, preferred_element_type=_F32)
  cg = jnp.sum(sel, axis=1, keepdims=True)

  @pl.when(i == 0)
  def _():
    pooled_ref[...] = jnp.zeros_like(pooled_ref)
    cg_ref[...] = jnp.zeros_like(cg_ref)

  pooled_ref[...] += psum
  cg_ref[...] += cg

  @pl.when(i == _NB - 1)
  def _():
    pooled_ref[...] = pooled_ref[...] / jnp.maximum(cg_ref[...], 1.0)


def kernel(x, edge_index, batch, W1l, b1l, W1r, W2l, b2l, W2r):
  src = edge_index[0]
  dst = edge_index[1]
  agg1, cnt = _sc_layer1(x, src, dst)
  w_spec = pl.BlockSpec((_D, _D), lambda i: (0, 0))
  b_spec = pl.BlockSpec((1, _D), lambda i: (0, 0))
  agg_spec = pl.BlockSpec((_NC, _R, _D), lambda i: (0, i, 0))
  cnt_spec = pl.BlockSpec((_NW, 1, 1, _R), lambda i: (0, i, 0, 0))
  cnt4 = cnt.reshape(_NW, _NB, 1, _R)
  row_spec = pl.BlockSpec((_R, _D), lambda i: (i, 0))
  h = pl.pallas_call(
      _tc1_body,
      grid=(_NB,),
      in_specs=[agg_spec, cnt_spec, row_spec, w_spec, b_spec, w_spec],
      out_specs=row_spec,
      out_shape=jax.ShapeDtypeStruct((_N, _D), _F32),
  )(agg1, cnt4, x, W1l, b1l.reshape(1, _D), W1r)
  (agg2,) = _sc_layer2(h, src, dst)
  pooled, h2 = pl.pallas_call(
      _tc2_body,
      grid=(_NB,),
      in_specs=[agg_spec, cnt_spec, row_spec, w_spec, b_spec, w_spec,
                pl.BlockSpec((1, 1, _R), lambda i: (i, 0, 0))],
      out_specs=(pl.BlockSpec((_G, _D), lambda i: (0, 0)), row_spec),
      out_shape=(jax.ShapeDtypeStruct((_G, _D), _F32),
                 jax.ShapeDtypeStruct((_N, _D), _F32)),
      scratch_shapes=[pltpu.VMEM((_G, 1), _F32)],
  )(agg2, cnt4, h, W2l, b2l.reshape(1, _D), W2r, batch.reshape(_NB, 1, _R))
  return (pooled, h2)


# speedup vs baseline: 12.8621x; 1.1159x over previous
import dataclasses
import functools

import jax
import jax.numpy as jnp
from jax import lax
from jax.experimental import pallas as pl
from jax.experimental.pallas import tpu as pltpu
from jax.experimental.pallas import tpu_sc as plsc

_N = 10000
_E = 320000
_D = 128
_G = 64

_NC = 2
_NS = 16
_NW = _NC * _NS
_EPW = _E // _NW
_CH = 80
_GC = 25
_NG = _EPW // (_GC * _CH)
_RSTRIDE = 624
_RWIN = 640
_ZR = 16
_L = 16

_F32 = jnp.float32


def _make_sc_msgpass(with_cnt):
  mesh = plsc.VectorSubcoreMesh(core_axis_name="c", subcore_axis_name="s")
  out_type = [jax.ShapeDtypeStruct((_NC, _N, _D), _F32)]
  scratch = [
      pltpu.VMEM_SHARED((_N, _D), _F32),
      pltpu.VMEM((_ZR, _D), _F32),
      pltpu.VMEM((_GC, _CH), jnp.int32),
      pltpu.VMEM((_GC, _CH), jnp.int32),
      pltpu.VMEM((_CH, _D), _F32),
      pltpu.VMEM((_CH, _D), _F32),
      pltpu.VMEM((_CH, _D), _F32),
      pltpu.SemaphoreType.DMA,
      pltpu.SemaphoreType.DMA,
      pltpu.SemaphoreType.DMA,
      pltpu.SemaphoreType.DMA,
      pltpu.SemaphoreType.DMA,
      pltpu.SemaphoreType.DMA,
  ]
  if with_cnt:
    out_type.append(jax.ShapeDtypeStruct((_NC, _NS, _N), _F32))
    scratch.append(pltpu.VMEM((_N,), _F32))
  cp = pltpu.CompilerParams()
  if "needs_layout_passes" in pltpu.CompilerParams.__dataclass_fields__:
    cp = dataclasses.replace(cp, needs_layout_passes=False)

  def body(x_hbm, src_hbm, dst_hbm, *rest):
    if with_cnt:
      (agg_hbm, cnt_hbm, agg_sh, zbuf, src_v, dst_v, rows_a, rows_b, rows_c,
       gsem_a, gsem_b, gsem_c, ssem_a, ssem_b, ssem_c, hist) = rest
    else:
      (agg_hbm, agg_sh, zbuf, src_v, dst_v, rows_a, rows_b, rows_c,
       gsem_a, gsem_b, gsem_c, ssem_a, ssem_b, ssem_c) = rest
    bufs = ((rows_a, gsem_a, ssem_a), (rows_b, gsem_b, ssem_b),
            (rows_c, gsem_c, ssem_c))
    c = lax.axis_index("c")
    s = lax.axis_index("s")

    @pl.loop(0, _ZR)
    def _(i):
      @pl.loop(0, _D // _L)
      def _(j):
        zbuf.at[i, pl.ds(j * _L, _L)][...] = jnp.zeros((_L,), _F32)

    if with_cnt:
      @pl.loop(0, _N // _L)
      def _(i):
        hist.at[pl.ds(i * _L, _L)][...] = jnp.zeros((_L,), _F32)

    rbase = s * _RSTRIDE

    @pl.loop(0, _RWIN // _ZR)
    def _(k):
      pltpu.sync_copy(zbuf, agg_sh.at[pl.ds(rbase + k * _ZR, _ZR)])

    plsc.subcore_barrier()

    wid = c * _NS + s

    def g_start(i, b):
      pltpu.async_copy(x_hbm.at[src_v.at[i]], bufs[b][0], bufs[b][1])

    def g_wait(i, b):
      pltpu.make_async_copy(x_hbm.at[src_v.at[i]], bufs[b][0],
                            bufs[b][1]).wait()

    def s_start(i, b):
      pltpu.async_copy(bufs[b][0], agg_sh.at[dst_v.at[i]], bufs[b][2],
                       add=True)
      if with_cnt:
        @pl.loop(0, _CH // _L)
        def _(k):
          ii = dst_v.at[i, pl.ds(k * _L, _L)][...]
          plsc.addupdate_scatter(hist, [ii], jnp.ones((_L,), _F32))

    def s_wait(i, b):
      pltpu.make_async_copy(bufs[b][0], agg_sh.at[dst_v.at[i]],
                            bufs[b][2]).wait()

    @pl.loop(0, _NG)
    def _(g):
      pltpu.sync_copy(src_hbm.at[wid, g], src_v)
      pltpu.sync_copy(dst_hbm.at[wid, g], dst_v)
      g_start(0, 0)
      g_start(1, 1)
      g_wait(0, 0)
      s_start(0, 0)
      g_start(2, 2)
      g_wait(1, 1)
      s_start(1, 1)

      @pl.loop(0, (_GC - 4) // 3)
      def _(jj):
        c0 = 3 * jj
        s_wait(c0, 0)
        g_start(c0 + 3, 0)
        g_wait(c0 + 2, 2)
        s_start(c0 + 2, 2)
        s_wait(c0 + 1, 1)
        g_start(c0 + 4, 1)
        g_wait(c0 + 3, 0)
        s_start(c0 + 3, 0)
        s_wait(c0 + 2, 2)
        g_start(c0 + 5, 2)
        g_wait(c0 + 4, 1)
        s_start(c0 + 4, 1)

      s_wait(_GC - 4, 0)
      g_start(_GC - 1, 0)
      g_wait(_GC - 2, 2)
      s_start(_GC - 2, 2)
      g_wait(_GC - 1, 0)
      s_start(_GC - 1, 0)
      s_wait(_GC - 3, 1)
      s_wait(_GC - 2, 2)
      s_wait(_GC - 1, 0)

    plsc.subcore_barrier()

    pltpu.sync_copy(agg_sh.at[pl.ds(rbase, _RWIN)],
                    agg_hbm.at[c, pl.ds(rbase, _RWIN)])
    if with_cnt:
      pltpu.sync_copy(hist, cnt_hbm.at[c, s])

  kern = pl.kernel(body, out_type=tuple(out_type), mesh=mesh,
                   scratch_types=scratch, compiler_params=cp)

  def call(x, src, dst):
    return kern(x, src.reshape(_NW, _NG, _GC, _CH),
                dst.reshape(_NW, _NG, _GC, _CH))

  return call


_sc_layer1 = _make_sc_msgpass(True)
_sc_layer2 = _make_sc_msgpass(False)

_HI = lax.Precision.HIGHEST
_R = 2000
_NB = _N // _R


def _cnt_col(cnt_ref):
  cnt = cnt_ref[:, 0, 0, :]
  return lax.dot_general(cnt, jnp.ones((_NW, 1), _F32),
                         (((0,), (0,)), ((), ())),
                         precision=_HI, preferred_element_type=_F32)


def _tc1_body(agg_ref, cnt_ref, x_ref, w1l_ref, b1l_ref, w1r_ref, h_ref):
  cnt = _cnt_col(cnt_ref)
  mean = (agg_ref[0] + agg_ref[1]) / jnp.maximum(cnt, 1.0)
  h = (jnp.dot(mean, w1l_ref[...], precision=_HI, preferred_element_type=_F32)
       + jnp.dot(x_ref[...], w1r_ref[...], precision=_HI,
                 preferred_element_type=_F32)
       + b1l_ref[...])
  h_ref[...] = jnp.maximum(h, 0.0)


def _tc2_body(agg_ref, cnt_ref, h_ref, w2l_ref, b2l_ref, w2r_ref, batch_ref,
              pooled_ref, h2_ref, cg_ref):
  i = pl.program_id(0)
  cnt = _cnt_col(cnt_ref)
  mean = (agg_ref[0] + agg_ref[1]) / jnp.maximum(cnt, 1.0)
  h2 = (jnp.dot(mean, w2l_ref[...], precision=_HI, preferred_element_type=_F32)
        + jnp.dot(h_ref[...], w2r_ref[...], precision=_HI,
                  preferred_element_type=_F32)
        + b2l_ref[...])
  h2_ref[...] = h2
  sel = (lax.broadcasted_iota(jnp.int32, (_G, _R), 0)
         == batch_ref[0]).astype(_F32)
  psum = jnp.dot(sel, h2, precision=_HI, preferred_element_type=_F32)
  cg = jnp.sum(sel, axis=1, keepdims=True)

  @pl.when(i == 0)
  def _():
    pooled_ref[...] = jnp.zeros_like(pooled_ref)
    cg_ref[...] = jnp.zeros_like(cg_ref)

  pooled_ref[...] += psum
  cg_ref[...] += cg

  @pl.when(i == _NB - 1)
  def _():
    pooled_ref[...] = pooled_ref[...] / jnp.maximum(cg_ref[...], 1.0)


def kernel(x, edge_index, batch, W1l, b1l, W1r, W2l, b2l, W2r):
  src = edge_index[0]
  dst = edge_index[1]
  agg1, cnt = _sc_layer1(x, src, dst)
  w_spec = pl.BlockSpec((_D, _D), lambda i: (0, 0))
  b_spec = pl.BlockSpec((1, _D), lambda i: (0, 0))
  agg_spec = pl.BlockSpec((_NC, _R, _D), lambda i: (0, i, 0))
  cnt_spec = pl.BlockSpec((_NW, 1, 1, _R), lambda i: (0, i, 0, 0))
  cnt4 = cnt.reshape(_NW, _NB, 1, _R)
  row_spec = pl.BlockSpec((_R, _D), lambda i: (i, 0))
  h = pl.pallas_call(
      _tc1_body,
      grid=(_NB,),
      in_specs=[agg_spec, cnt_spec, row_spec, w_spec, b_spec, w_spec],
      out_specs=row_spec,
      out_shape=jax.ShapeDtypeStruct((_N, _D), _F32),
  )(agg1, cnt4, x, W1l, b1l.reshape(1, _D), W1r)
  (agg2,) = _sc_layer2(h, src, dst)
  pooled, h2 = pl.pallas_call(
      _tc2_body,
      grid=(_NB,),
      in_specs=[agg_spec, cnt_spec, row_spec, w_spec, b_spec, w_spec,
                pl.BlockSpec((1, 1, _R), lambda i: (i, 0, 0))],
      out_specs=(pl.BlockSpec((_G, _D), lambda i: (0, 0)), row_spec),
      out_shape=(jax.ShapeDtypeStruct((_G, _D), _F32),
                 jax.ShapeDtypeStruct((_N, _D), _F32)),
      scratch_shapes=[pltpu.VMEM((_G, 1), _F32)],
  )(agg2, cnt4, h, W2l, b2l.reshape(1, _D), W2r, batch.reshape(_NB, 1, _R))
  return (pooled, h2)


# speedup vs baseline: 12.9049x; 1.0033x over previous
import dataclasses
import functools

import jax
import jax.numpy as jnp
from jax import lax
from jax.experimental import pallas as pl
from jax.experimental.pallas import tpu as pltpu
from jax.experimental.pallas import tpu_sc as plsc

_N = 10000
_E = 320000
_D = 128
_G = 64

_NC = 2
_NS = 16
_NW = _NC * _NS
_EPW = _E // _NW
_CH = 80
_GC = 25
_NG = _EPW // (_GC * _CH)
_RSTRIDE = 624
_RWIN = 640
_ZR = 16
_L = 16

_F32 = jnp.float32


def _make_sc_msgpass(with_cnt):
  mesh = plsc.VectorSubcoreMesh(core_axis_name="c", subcore_axis_name="s")
  out_type = [jax.ShapeDtypeStruct((_NC, _N, _D), _F32)]
  scratch = [
      pltpu.VMEM_SHARED((_N, _D), _F32),
      pltpu.VMEM((_ZR, _D), _F32),
      pltpu.VMEM((_GC, _CH), jnp.int32),
      pltpu.VMEM((_GC, _CH), jnp.int32),
      pltpu.VMEM((_CH, _D), _F32),
      pltpu.VMEM((_CH, _D), _F32),
      pltpu.VMEM((_CH, _D), _F32),
      pltpu.SemaphoreType.DMA,
      pltpu.SemaphoreType.DMA,
      pltpu.SemaphoreType.DMA,
      pltpu.SemaphoreType.DMA,
      pltpu.SemaphoreType.DMA,
      pltpu.SemaphoreType.DMA,
  ]
  if with_cnt:
    out_type.append(jax.ShapeDtypeStruct((_NC, _NS, _N), _F32))
    scratch.append(pltpu.VMEM((_N,), _F32))
  cp = pltpu.CompilerParams()
  if "needs_layout_passes" in pltpu.CompilerParams.__dataclass_fields__:
    cp = dataclasses.replace(cp, needs_layout_passes=False)

  def body(x_hbm, src_hbm, dst_hbm, *rest):
    if with_cnt:
      (agg_hbm, cnt_hbm, agg_sh, zbuf, src_v, dst_v, rows_a, rows_b, rows_c,
       gsem_a, gsem_b, gsem_c, ssem_a, ssem_b, ssem_c, hist) = rest
    else:
      (agg_hbm, agg_sh, zbuf, src_v, dst_v, rows_a, rows_b, rows_c,
       gsem_a, gsem_b, gsem_c, ssem_a, ssem_b, ssem_c) = rest
    bufs = ((rows_a, gsem_a, ssem_a), (rows_b, gsem_b, ssem_b),
            (rows_c, gsem_c, ssem_c))
    c = lax.axis_index("c")
    s = lax.axis_index("s")

    @pl.loop(0, _ZR)
    def _(i):
      @pl.loop(0, _D // _L)
      def _(j):
        zbuf.at[i, pl.ds(j * _L, _L)][...] = jnp.zeros((_L,), _F32)

    if with_cnt:
      @pl.loop(0, _N // _L)
      def _(i):
        hist.at[pl.ds(i * _L, _L)][...] = jnp.zeros((_L,), _F32)

    rbase = s * _RSTRIDE

    @pl.loop(0, _RWIN // _ZR)
    def _(k):
      pltpu.sync_copy(zbuf, agg_sh.at[pl.ds(rbase + k * _ZR, _ZR)])

    plsc.subcore_barrier()

    wid = c * _NS + s

    def g_start(i, b):
      pltpu.async_copy(x_hbm.at[src_v.at[i]], bufs[b][0], bufs[b][1])

    def g_wait(i, b):
      pltpu.make_async_copy(x_hbm.at[src_v.at[i]], bufs[b][0],
                            bufs[b][1]).wait()

    def s_start(i, b):
      pltpu.async_copy(bufs[b][0], agg_sh.at[dst_v.at[i]], bufs[b][2],
                       add=True)
      if with_cnt:
        @pl.loop(0, _CH // _L)
        def _(k):
          ii = dst_v.at[i, pl.ds(k * _L, _L)][...]
          plsc.addupdate_scatter(hist, [ii], jnp.ones((_L,), _F32))

    def s_wait(i, b):
      pltpu.make_async_copy(bufs[b][0], agg_sh.at[dst_v.at[i]],
                            bufs[b][2]).wait()

    @pl.loop(0, _NG)
    def _(g):
      pltpu.sync_copy(src_hbm.at[wid, g], src_v)
      pltpu.sync_copy(dst_hbm.at[wid, g], dst_v)
      g_start(0, 0)
      g_start(1, 1)
      g_wait(0, 0)
      s_start(0, 0)
      g_start(2, 2)
      g_wait(1, 1)
      s_start(1, 1)

      @pl.loop(0, (_GC - 4) // 3)
      def _(jj):
        c0 = 3 * jj
        s_wait(c0, 0)
        g_start(c0 + 3, 0)
        g_wait(c0 + 2, 2)
        s_start(c0 + 2, 2)
        s_wait(c0 + 1, 1)
        g_start(c0 + 4, 1)
        g_wait(c0 + 3, 0)
        s_start(c0 + 3, 0)
        s_wait(c0 + 2, 2)
        g_start(c0 + 5, 2)
        g_wait(c0 + 4, 1)
        s_start(c0 + 4, 1)

      s_wait(_GC - 4, 0)
      g_start(_GC - 1, 0)
      g_wait(_GC - 2, 2)
      s_start(_GC - 2, 2)
      g_wait(_GC - 1, 0)
      s_start(_GC - 1, 0)
      s_wait(_GC - 3, 1)
      s_wait(_GC - 2, 2)
      s_wait(_GC - 1, 0)

    plsc.subcore_barrier()

    pltpu.sync_copy(agg_sh.at[pl.ds(rbase, _RWIN)],
                    agg_hbm.at[c, pl.ds(rbase, _RWIN)])
    if with_cnt:
      pltpu.sync_copy(hist, cnt_hbm.at[c, s])

  kern = pl.kernel(body, out_type=tuple(out_type), mesh=mesh,
                   scratch_types=scratch, compiler_params=cp)

  def call(x, src, dst):
    return kern(x, src.reshape(_NW, _NG, _GC, _CH),
                dst.reshape(_NW, _NG, _GC, _CH))

  return call


_sc_layer1 = _make_sc_msgpass(True)
_sc_layer2 = _make_sc_msgpass(False)

_HI = lax.Precision.HIGHEST
_R = 2000
_NB = _N // _R


def _cnt_col(cnt_ref):
  cnt = cnt_ref[:, 0, 0, :]
  return lax.dot_general(cnt, jnp.ones((_NW, 1), _F32),
                         (((0,), (0,)), ((), ())),
                         precision=_HI, preferred_element_type=_F32)


def _root_body(x_ref, w_ref, b_ref, out_ref):
  out_ref[...] = (jnp.dot(x_ref[...], w_ref[...], precision=_HI,
                          preferred_element_type=_F32) + b_ref[...])


def _tc1_body(agg_ref, cnt_ref, xr_ref, w1l_ref, h_ref):
  cnt = _cnt_col(cnt_ref)
  mean = (agg_ref[0] + agg_ref[1]) / jnp.maximum(cnt, 1.0)
  h = (jnp.dot(mean, w1l_ref[...], precision=_HI, preferred_element_type=_F32)
       + xr_ref[...])
  h_ref[...] = jnp.maximum(h, 0.0)


def _tc2_body(agg_ref, cnt_ref, hr_ref, w2l_ref, batch_ref,
              pooled_ref, h2_ref, cg_ref):
  i = pl.program_id(0)
  cnt = _cnt_col(cnt_ref)
  mean = (agg_ref[0] + agg_ref[1]) / jnp.maximum(cnt, 1.0)
  h2 = (jnp.dot(mean, w2l_ref[...], precision=_HI, preferred_element_type=_F32)
        + hr_ref[...])
  h2_ref[...] = h2
  sel = (lax.broadcasted_iota(jnp.int32, (_G, _R), 0)
         == batch_ref[0]).astype(_F32)
  psum = jnp.dot(sel, h2, precision=_HI, preferred_element_type=_F32)
  cg = jnp.sum(sel, axis=1, keepdims=True)

  @pl.when(i == 0)
  def _():
    pooled_ref[...] = jnp.zeros_like(pooled_ref)
    cg_ref[...] = jnp.zeros_like(cg_ref)

  pooled_ref[...] += psum
  cg_ref[...] += cg

  @pl.when(i == _NB - 1)
  def _():
    pooled_ref[...] = pooled_ref[...] / jnp.maximum(cg_ref[...], 1.0)


_w_spec = pl.BlockSpec((_D, _D), lambda i: (0, 0))
_b_spec = pl.BlockSpec((1, _D), lambda i: (0, 0))
_agg_spec = pl.BlockSpec((_NC, _R, _D), lambda i: (0, i, 0))
_cnt_spec = pl.BlockSpec((_NW, 1, 1, _R), lambda i: (0, i, 0, 0))
_row_spec = pl.BlockSpec((_R, _D), lambda i: (i, 0))


def _root_mm(x, W, b):
  return pl.pallas_call(
      _root_body,
      grid=(_NB,),
      in_specs=[_row_spec, _w_spec, _b_spec],
      out_specs=_row_spec,
      out_shape=jax.ShapeDtypeStruct((_N, _D), _F32),
  )(x, W, b.reshape(1, _D))


def kernel(x, edge_index, batch, W1l, b1l, W1r, W2l, b2l, W2r):
  src = edge_index[0]
  dst = edge_index[1]
  agg1, cnt = _sc_layer1(x, src, dst)
  xr = _root_mm(x, W1r, b1l)
  cnt4 = cnt.reshape(_NW, _NB, 1, _R)
  h = pl.pallas_call(
      _tc1_body,
      grid=(_NB,),
      in_specs=[_agg_spec, _cnt_spec, _row_spec, _w_spec],
      out_specs=_row_spec,
      out_shape=jax.ShapeDtypeStruct((_N, _D), _F32),
  )(agg1, cnt4, xr, W1l)
  (agg2,) = _sc_layer2(h, src, dst)
  hr = _root_mm(h, W2r, b2l)
  pooled, h2 = pl.pallas_call(
      _tc2_body,
      grid=(_NB,),
      in_specs=[_agg_spec, _cnt_spec, _row_spec, _w_spec,
                pl.BlockSpec((1, 1, _R), lambda i: (i, 0, 0))],
      out_specs=(pl.BlockSpec((_G, _D), lambda i: (0, 0)), _row_spec),
      out_shape=(jax.ShapeDtypeStruct((_G, _D), _F32),
                 jax.ShapeDtypeStruct((_N, _D), _F32)),
      scratch_shapes=[pltpu.VMEM((_G, 1), _F32)],
  )(agg2, cnt4, hr, W2l, batch.reshape(_NB, 1, _R))
  return (pooled, h2)


# speedup vs baseline: 12.9206x; 1.0012x over previous
import dataclasses
import functools

import jax
import jax.numpy as jnp
from jax import lax
from jax.experimental import pallas as pl
from jax.experimental.pallas import tpu as pltpu
from jax.experimental.pallas import tpu_sc as plsc

_N = 10000
_E = 320000
_D = 128
_G = 64

_NC = 2
_NS = 16
_NW = _NC * _NS
_EPW = _E // _NW
_CH = 80
_M = _EPW // _CH
_RSTRIDE = 624
_RWIN = 640
_ZR = 16
_L = 16

_F32 = jnp.float32


def _make_sc_msgpass(with_cnt):
  mesh = plsc.VectorSubcoreMesh(core_axis_name="c", subcore_axis_name="s")
  out_type = [jax.ShapeDtypeStruct((_NC, _N, _D), _F32)]
  scratch = [
      pltpu.VMEM_SHARED((_N, _D), _F32),
      pltpu.VMEM((_ZR, _D), _F32),
  ]
  for _ in range(3):
    scratch += [
        pltpu.VMEM((_CH,), jnp.int32),
        pltpu.VMEM((_CH,), jnp.int32),
        pltpu.VMEM((_CH, _D), _F32),
        pltpu.SemaphoreType.DMA,
        pltpu.SemaphoreType.DMA,
        pltpu.SemaphoreType.DMA,
    ]
  if with_cnt:
    out_type.append(jax.ShapeDtypeStruct((_NC, _NS, _N), _F32))
    scratch.append(pltpu.VMEM((_N,), _F32))
  cp = pltpu.CompilerParams()
  if "needs_layout_passes" in pltpu.CompilerParams.__dataclass_fields__:
    cp = dataclasses.replace(cp, needs_layout_passes=False)

  def body(x_hbm, edge_hbm, *rest):
    if with_cnt:
      (agg_hbm, cnt_hbm, agg_sh, zbuf, *ring, hist) = rest
    else:
      (agg_hbm, agg_sh, zbuf, *ring) = rest
    bufs = (tuple(ring[0:6]), tuple(ring[6:12]), tuple(ring[12:18]))
    c = lax.axis_index("c")
    s = lax.axis_index("s")

    @pl.loop(0, _ZR)
    def _(i):
      @pl.loop(0, _D // _L)
      def _(j):
        zbuf.at[i, pl.ds(j * _L, _L)][...] = jnp.zeros((_L,), _F32)

    if with_cnt:
      @pl.loop(0, _N // _L)
      def _(i):
        hist.at[pl.ds(i * _L, _L)][...] = jnp.zeros((_L,), _F32)

    rbase = s * _RSTRIDE

    @pl.loop(0, _RWIN // _ZR)
    def _(k):
      pltpu.sync_copy(zbuf, agg_sh.at[pl.ds(rbase + k * _ZR, _ZR)])

    plsc.subcore_barrier()

    wid = c * _NS + s
    ebase = wid * _EPW

    def i_start(ch, b):
      src_v, dst_v, _, isem, _, _ = bufs[b]
      off = ebase + ch * _CH
      pltpu.async_copy(edge_hbm.at[pl.ds(off, _CH)], src_v, isem)
      pltpu.async_copy(edge_hbm.at[pl.ds(_E + off, _CH)], dst_v, isem)

    def i_wait(ch, b):
      src_v, dst_v, _, isem, _, _ = bufs[b]
      off = ebase + ch * _CH
      pltpu.make_async_copy(edge_hbm.at[pl.ds(off, _CH)], src_v,
                            isem).wait()
      pltpu.make_async_copy(edge_hbm.at[pl.ds(_E + off, _CH)], dst_v,
                            isem).wait()

    def g_start(b):
      src_v, _, rows, _, gsem, _ = bufs[b]
      pltpu.async_copy(x_hbm.at[src_v], rows, gsem)

    def g_wait(b):
      src_v, _, rows, _, gsem, _ = bufs[b]
      pltpu.make_async_copy(x_hbm.at[src_v], rows, gsem).wait()

    def s_start(b):
      _, dst_v, rows, _, _, ssem = bufs[b]
      pltpu.async_copy(rows, agg_sh.at[dst_v], ssem,
                       add=True)
      if with_cnt:
        @pl.loop(0, _CH // _L)
        def _(k):
          ii = dst_v.at[pl.ds(k * _L, _L)][...]
          plsc.addupdate_scatter(hist, [ii], jnp.ones((_L,), _F32))

    def s_wait(b):
      _, dst_v, rows, _, _, ssem = bufs[b]
      pltpu.make_async_copy(rows, agg_sh.at[dst_v], ssem).wait()

    i_start(0, 0)
    i_start(1, 1)
    i_wait(0, 0)
    g_start(0)
    i_start(2, 2)
    i_wait(1, 1)
    g_start(1)
    g_wait(0)
    s_start(0)

    @pl.loop(0, (_M - 5) // 3)
    def _(jj):
      c0 = 3 * jj
      s_wait(0)
      i_start(c0 + 3, 0)
      i_wait(c0 + 2, 2)
      g_start(2)
      g_wait(1)
      s_start(1)

      s_wait(1)
      i_start(c0 + 4, 1)
      i_wait(c0 + 3, 0)
      g_start(0)
      g_wait(2)
      s_start(2)

      s_wait(2)
      i_start(c0 + 5, 2)
      i_wait(c0 + 4, 1)
      g_start(1)
      g_wait(0)
      s_start(0)

    s_wait(0)
    i_start(_M - 2, 0)
    i_wait(_M - 3, 2)
    g_start(2)
    g_wait(1)
    s_start(1)

    s_wait(1)
    i_start(_M - 1, 1)
    i_wait(_M - 2, 0)
    g_start(0)
    g_wait(2)
    s_start(2)

    s_wait(2)
    i_wait(_M - 1, 1)
    g_start(1)
    g_wait(0)
    s_start(0)

    g_wait(1)
    s_start(1)

    s_wait(0)
    s_wait(1)

    plsc.subcore_barrier()

    pltpu.sync_copy(agg_sh.at[pl.ds(rbase, _RWIN)],
                    agg_hbm.at[c, pl.ds(rbase, _RWIN)])
    if with_cnt:
      pltpu.sync_copy(hist, cnt_hbm.at[c, s])

  kern = pl.kernel(body, out_type=tuple(out_type), mesh=mesh,
                   scratch_types=scratch, compiler_params=cp)

  return kern


_sc_layer1 = _make_sc_msgpass(True)
_sc_layer2 = _make_sc_msgpass(False)

_HI = lax.Precision.HIGHEST
_R = 2000
_NB = _N // _R


def _cnt_col(cnt_ref, i):
  del i
  cnt = cnt_ref[:, 0, 0, :]
  return lax.dot_general(cnt, jnp.ones((_NW, 1), _F32),
                         (((0,), (0,)), ((), ())),
                         precision=_HI, preferred_element_type=_F32)


def _root_body(x_ref, w_ref, b_ref, out_ref):
  out_ref[...] = (jnp.dot(x_ref[...], w_ref[...], precision=_HI,
                          preferred_element_type=_F32) + b_ref[...])


def _tc1_body(agg_ref, cnt_ref, xr_ref, w1l_ref, h_ref):
  cnt = _cnt_col(cnt_ref, pl.program_id(0))
  mean = (agg_ref[0] + agg_ref[1]) / jnp.maximum(cnt, 1.0)
  h = (jnp.dot(mean, w1l_ref[...], precision=_HI, preferred_element_type=_F32)
       + xr_ref[...])
  h_ref[...] = jnp.maximum(h, 0.0)


def _tc2_body(agg_ref, cnt_ref, hr_ref, w2l_ref, batch_ref,
              pooled_ref, h2_ref, cg_ref):
  i = pl.program_id(0)
  cnt = _cnt_col(cnt_ref, i)
  mean = (agg_ref[0] + agg_ref[1]) / jnp.maximum(cnt, 1.0)
  h2 = (jnp.dot(mean, w2l_ref[...], precision=_HI, preferred_element_type=_F32)
        + hr_ref[...])
  h2_ref[...] = h2
  sel = (lax.broadcasted_iota(jnp.int32, (_G, _R), 0)
         == batch_ref[0]).astype(_F32)
  psum = jnp.dot(sel, h2, precision=_HI, preferred_element_type=_F32)
  cg = jnp.sum(sel, axis=1, keepdims=True)

  @pl.when(i == 0)
  def _():
    pooled_ref[...] = jnp.zeros_like(pooled_ref)
    cg_ref[...] = jnp.zeros_like(cg_ref)

  pooled_ref[...] += psum
  cg_ref[...] += cg

  @pl.when(i == _NB - 1)
  def _():
    pooled_ref[...] = pooled_ref[...] / jnp.maximum(cg_ref[...], 1.0)


_w_spec = pl.BlockSpec((_D, _D), lambda i: (0, 0))
_b_spec = pl.BlockSpec((1, _D), lambda i: (0, 0))
_agg_spec = pl.BlockSpec((_NC, _R, _D), lambda i: (0, i, 0))
_cnt_spec = pl.BlockSpec((_NW, 1, 1, _R), lambda i: (0, i, 0, 0))
_row_spec = pl.BlockSpec((_R, _D), lambda i: (i, 0))


def _root_mm(x, W, b):
  return pl.pallas_call(
      _root_body,
      grid=(_NB,),
      in_specs=[_row_spec, _w_spec, _b_spec],
      out_specs=_row_spec,
      out_shape=jax.ShapeDtypeStruct((_N, _D), _F32),
  )(x, W, b.reshape(1, _D))


def kernel(x, edge_index, batch, W1l, b1l, W1r, W2l, b2l, W2r):
  edge_flat = edge_index.reshape(2 * _E)
  agg1, cnt = _sc_layer1(x, edge_flat)
  xr = _root_mm(x, W1r, b1l)
  h = pl.pallas_call(
      _tc1_body,
      grid=(_NB,),
      in_specs=[_agg_spec, _cnt_spec, _row_spec, _w_spec],
      out_specs=_row_spec,
      out_shape=jax.ShapeDtypeStruct((_N, _D), _F32),
  )(agg1, cnt.reshape(_NW, _NB, 1, _R), xr, W1l)
  (agg2,) = _sc_layer2(h, edge_flat)
  hr = _root_mm(h, W2r, b2l)
  pooled, h2 = pl.pallas_call(
      _tc2_body,
      grid=(_NB,),
      in_specs=[_agg_spec, _cnt_spec, _row_spec, _w_spec,
                pl.BlockSpec((1, 1, _R), lambda i: (i, 0, 0))],
      out_specs=(pl.BlockSpec((_G, _D), lambda i: (0, 0)), _row_spec),
      out_shape=(jax.ShapeDtypeStruct((_G, _D), _F32),
                 jax.ShapeDtypeStruct((_N, _D), _F32)),
      scratch_shapes=[pltpu.VMEM((_G, 1), _F32)],
  )(agg2, cnt.reshape(_NW, _NB, 1, _R), hr, W2l, batch.reshape(_NB, 1, _R))
  return (pooled, h2)
